# Initial kernel scaffold; baseline (speedup 1.0000x reference)
import jax
import jax.numpy as jnp
from jax.experimental import pallas as pl


def kernel(idx, token_table, pos_table):
    raise NotImplementedError("write your pallas kernel here")


# speedup vs baseline: 1.2707x; 1.2707x over previous
import functools

import jax
import jax.numpy as jnp
from jax import lax
from jax.experimental import pallas as pl
from jax.experimental.pallas import tpu as pltpu
from jax.experimental.pallas import tpu_sc as plsc

NC = 2
NS = 16
NW = NC * NS
LANES = 16
IDX_CHUNK = 128


@functools.lru_cache(maxsize=None)
def _build(B, T, D):
    total = B * T
    rpw = total // NW
    n_gather = rpw // IDX_CHUNK
    mesh = plsc.VectorSubcoreMesh(core_axis_name="c", subcore_axis_name="s")

    @functools.partial(
        pl.kernel,
        out_type=jax.ShapeDtypeStruct((total, D), jnp.float32),
        mesh=mesh,
        scratch_types=[
            pltpu.VMEM((n_gather, IDX_CHUNK), jnp.int32),
            pltpu.VMEM((rpw, D), jnp.float32),
            pltpu.VMEM((rpw, D), jnp.float32),
            pltpu.SemaphoreType.DMA,
        ],
    )
    def sc_kernel(idx_hbm, tok_hbm, pos_hbm, out_hbm, idx_v, rows_v, pos_v, sem):
        c = lax.axis_index("c")
        s = lax.axis_index("s")
        wid = s * NC + c
        base = wid * rpw
        tbase = lax.rem(base, T)

        pltpu.sync_copy(idx_hbm.at[wid], idx_v)

        copies = []
        for g in range(n_gather):
            copies.append(
                pltpu.async_copy(
                    tok_hbm.at[idx_v.at[g]],
                    rows_v.at[pl.ds(g * IDX_CHUNK, IDX_CHUNK)],
                    sem,
                )
            )
        pltpu.sync_copy(pos_hbm.at[pl.ds(tbase, rpw)], pos_v)
        for cp in copies:
            cp.wait()

        def add_row(r, carry):
            for ch in range(D // LANES):
                sl = pl.ds(ch * LANES, LANES)
                rows_v[r, sl] = rows_v[r, sl] + pos_v[r, sl]
            return carry

        lax.fori_loop(0, rpw, add_row, 0)

        pltpu.sync_copy(rows_v, out_hbm.at[pl.ds(base, rpw)])

    return sc_kernel


def kernel(idx, token_table, pos_table):
    B, T = idx.shape
    V, D = token_table.shape
    total = B * T
    rpw = total // NW
    assert total % NW == 0 and rpw % IDX_CHUNK == 0 and D % LANES == 0
    assert T % rpw == 0

    idx3 = idx.astype(jnp.int32).reshape(NW, rpw // IDX_CHUNK, IDX_CHUNK)
    out = _build(B, T, D)(idx3, token_table, pos_table)
    return out.reshape(B, T, D)


# speedup vs baseline: 1.3474x; 1.0603x over previous
import functools

import jax
import jax.numpy as jnp
from jax import lax
from jax.experimental import pallas as pl
from jax.experimental.pallas import tpu as pltpu
from jax.experimental.pallas import tpu_sc as plsc

NC = 2
NS = 16
NW = NC * NS
LANES = 16


@functools.lru_cache(maxsize=None)
def _build(B, T, D):
    tpw = T // NW
    mesh = plsc.VectorSubcoreMesh(core_axis_name="c", subcore_axis_name="s")

    @functools.partial(
        pl.kernel,
        out_type=jax.ShapeDtypeStruct((B * T, D), jnp.float32),
        mesh=mesh,
        scratch_types=[
            pltpu.VMEM((B, tpw), jnp.int32),
            pltpu.VMEM((B, tpw, D), jnp.float32),
            pltpu.VMEM((tpw, D), jnp.float32),
            pltpu.SemaphoreType.DMA((B,)),
            pltpu.SemaphoreType.DMA((B,)),
        ],
    )
    def sc_kernel(idx_hbm, tok_hbm, pos_hbm, out_hbm, idx_v, rows_v, pos_v,
                  gsem, osem):
        c = lax.axis_index("c")
        s = lax.axis_index("s")
        wid = s * NC + c
        tbase = wid * tpw

        pltpu.sync_copy(idx_hbm.at[wid], idx_v)

        gathers = [
            pltpu.async_copy(tok_hbm.at[idx_v.at[b]], rows_v.at[b], gsem.at[b])
            for b in range(B)
        ]
        pltpu.sync_copy(pos_hbm.at[pl.ds(tbase, tpw)], pos_v)

        stores = []
        for b in range(B):
            gathers[b].wait()

            def add_row(r, carry, b=b):
                for ch in range(D // LANES):
                    sl = pl.ds(ch * LANES, LANES)
                    rows_v[b, r, sl] = rows_v[b, r, sl] + pos_v[r, sl]
                return carry

            lax.fori_loop(0, tpw, add_row, 0)
            stores.append(
                pltpu.async_copy(
                    rows_v.at[b], out_hbm.at[pl.ds(b * T + tbase, tpw)],
                    osem.at[b],
                )
            )
        for st in stores:
            st.wait()

    return sc_kernel


def kernel(idx, token_table, pos_table):
    B, T = idx.shape
    V, D = token_table.shape
    tpw = T // NW
    assert T % NW == 0 and tpw % 8 == 0 and tpw <= 128 and D % LANES == 0

    idx_r = idx.astype(jnp.int32).reshape(B, NW, tpw).transpose(1, 0, 2)
    out = _build(B, T, D)(idx_r, token_table, pos_table)
    return out.reshape(B, T, D)
